# baseline (speedup 1.0000x reference)
import functools

import jax
import jax.numpy as jnp
from jax import lax
from jax.experimental import pallas as pl
from jax.experimental.pallas import tpu as pltpu
from jax.experimental.pallas import tpu_sc as plsc

T = 8
N = 512
F = 128
H1 = 64
H2 = 8
E = 8192
NH = N * H2

_NC = 2
_NS = 16
_TS_PER_CORE = T // _NC
_CHUNKS = _NS // _TS_PER_CORE
_EC = E // _CHUNKS
_A_WORDS = N * N
_SH_WORDS = _TS_PER_CORE * _A_WORDS
_ZW = _SH_WORDS // _NS
_SCAT = 128
_NSCAT = _EC // _SCAT
_NIDX = _EC // 16


_ZB = 8192


def _adj_body(edges, weights, out, src_v, dst_v, w_v, idx2, w2, zbuf, a_sh):
    c = lax.axis_index("c")
    s = lax.axis_index("s")
    ts_local = s // _CHUNKS
    ts = c * _TS_PER_CORE + ts_local
    chunk = s % _CHUNKS

    def zfill(i, _):
        zbuf[pl.ds(i * 16, 16)] = jnp.zeros((16,), jnp.float32)
        return 0

    lax.fori_loop(0, _ZB // 16, zfill, 0)

    def zcopy(i, _):
        pltpu.sync_copy(zbuf, a_sh.at[pl.ds(s * _ZW + i * _ZB, _ZB)])
        return 0

    lax.fori_loop(0, _ZW // _ZB, zcopy, 0)
    pltpu.sync_copy(edges.at[ts, 0, pl.ds(chunk * _EC, _EC)], src_v)
    pltpu.sync_copy(edges.at[ts, 1, pl.ds(chunk * _EC, _EC)], dst_v)
    pltpu.sync_copy(weights.at[ts, pl.ds(chunk * _EC, _EC)], w_v)

    base = ts_local * _A_WORDS

    def idx_body(i, _):
        d = dst_v[pl.ds(i * 16, 16)]
        sr = src_v[pl.ds(i * 16, 16)]
        ww = w_v[pl.ds(i * 16, 16)]
        ix = base + d * N + sr
        row = i // (_SCAT // 16)
        col = (i % (_SCAT // 16)) * 16
        idx2[row, pl.ds(col, 16)] = ix
        w2[row, pl.ds(col, 16)] = ww
        return 0

    lax.fori_loop(0, _NIDX, idx_body, 0)
    plsc.subcore_barrier()

    def scat_body(j, _):
        pltpu.sync_copy(w2.at[j], a_sh.at[idx2.at[j]], add=True)
        return 0

    lax.fori_loop(0, _NSCAT, scat_body, 0)
    plsc.subcore_barrier()

    out_row = c * _TS_PER_CORE + s // _CHUNKS
    out_col = (s % _CHUNKS) * _ZW
    pltpu.sync_copy(a_sh.at[pl.ds(s * _ZW, _ZW)],
                    out.at[out_row, pl.ds(out_col, _ZW)])


def _build_adj(edge_index, edge_weight):
    mesh = plsc.VectorSubcoreMesh(core_axis_name="c", subcore_axis_name="s")
    k = pl.kernel(
        _adj_body,
        mesh=mesh,
        out_type=jax.ShapeDtypeStruct((T, _A_WORDS), jnp.float32),
        scratch_types=[
            pltpu.VMEM((_EC,), jnp.int32),
            pltpu.VMEM((_EC,), jnp.int32),
            pltpu.VMEM((_EC,), jnp.float32),
            pltpu.VMEM((_NSCAT, _SCAT), jnp.int32),
            pltpu.VMEM((_NSCAT, _SCAT), jnp.float32),
            pltpu.VMEM((_ZB,), jnp.float32),
            pltpu.VMEM_SHARED((_SH_WORDS,), jnp.float32),
        ],
    )
    return k(edge_index, edge_weight)


def _xw1_body(x_ref, w1_ref, o_ref):
    o_ref[0] = jnp.dot(x_ref[0], w1_ref[0], preferred_element_type=jnp.float32)


def _xw1(x, W1):
    return pl.pallas_call(
        _xw1_body,
        grid=(T,),
        in_specs=[
            pl.BlockSpec((1, N, F), lambda i: (i, 0, 0)),
            pl.BlockSpec((1, F, H1), lambda i: (i, 0, 0)),
        ],
        out_specs=pl.BlockSpec((1, N, H1), lambda i: (i, 0, 0)),
        out_shape=jax.ShapeDtypeStruct((T, N, H1), jnp.float32),
    )(x, W1)


def _gcn_body(a_ref, xw1_ref, n_ref, w2_ref, rec_ref, z_ref):
    a = a_ref[0]
    h = jnp.maximum(jnp.dot(a, xw1_ref[0], preferred_element_type=jnp.float32),
                    0.0)
    h = h + 0.1 * n_ref[0]
    hw2 = jnp.dot(h, w2_ref[0], preferred_element_type=jnp.float32)
    z = jnp.dot(a, hw2, preferred_element_type=jnp.float32)
    z_ref[0] = z
    rec_ref[0] = lax.dot_general(z, z, (((1,), (1,)), ((), ())),
                                 preferred_element_type=jnp.float32)


def _gcn(adj, xw1, noise, W2):
    return pl.pallas_call(
        _gcn_body,
        grid=(T,),
        in_specs=[
            pl.BlockSpec((1, N, N), lambda i: (i, 0, 0)),
            pl.BlockSpec((1, N, H1), lambda i: (i, 0, 0)),
            pl.BlockSpec((1, N, H1), lambda i: (i, 0, 0)),
            pl.BlockSpec((1, H1, H2), lambda i: (i, 0, 0)),
        ],
        out_specs=[
            pl.BlockSpec((1, N, N), lambda i: (i, 0, 0)),
            pl.BlockSpec((1, N, H2), lambda i: (i, 0, 0)),
        ],
        out_shape=[
            jax.ShapeDtypeStruct((T, N, N), jnp.float32),
            jax.ShapeDtypeStruct((T, N, H2), jnp.float32),
        ],
    )(adj, xw1, noise, W2)


_CB = 512
_NB = NH // _CB


def _tcn_body(s_ref, x2_ref, w_ref, b_ref, y_ref):
    i = pl.program_id(0)
    conv = b_ref[...] + jnp.dot(x2_ref[...], w_ref[...],
                                preferred_element_type=jnp.float32)
    y = jnp.maximum(conv, 0.0)
    res = s_ref[:, pl.ds(i * _CB, _CB)]
    y_ref[...] = jnp.maximum(y + res, 0.0)


def _tcn_stage(s, x2, w_cat, b):
    return pl.pallas_call(
        _tcn_body,
        grid=(_NB,),
        in_specs=[
            pl.BlockSpec((T, NH), lambda i: (0, 0)),
            pl.BlockSpec((T, 2 * NH), lambda i: (0, 0)),
            pl.BlockSpec((2 * NH, _CB), lambda i: (0, i)),
            pl.BlockSpec((1, _CB), lambda i: (0, i)),
        ],
        out_specs=pl.BlockSpec((T, _CB), lambda i: (0, i)),
        out_shape=jax.ShapeDtypeStruct((T, NH), jnp.float32),
    )(s, x2, w_cat, b)


def _outer_body(rec0_ref, z_ref, r_ref):
    z = z_ref[0]
    r_ref[0, 0] = rec0_ref[0]
    r_ref[1, 0] = lax.dot_general(z, z, (((1,), (1,)), ((), ())),
                                  preferred_element_type=jnp.float32)


def _outer(rec0, z):
    return pl.pallas_call(
        _outer_body,
        grid=(T,),
        in_specs=[
            pl.BlockSpec((1, N, N), lambda i: (i, 0, 0)),
            pl.BlockSpec((1, N, H2), lambda i: (i, 0, 0)),
        ],
        out_specs=pl.BlockSpec((2, 1, N, N), lambda i: (0, i, 0, 0)),
        out_shape=jax.ShapeDtypeStruct((2, T, N, N), jnp.float32),
    )(rec0, z)


def kernel(struct_features, edge_index, edge_weight, noise, W1, W2,
           tcn_w0, tcn_b0, tcn_w1, tcn_b1):
    edge_index = edge_index.astype(jnp.int32)
    edge_weight = edge_weight.astype(jnp.float32)

    adj = _build_adj(edge_index, edge_weight).reshape(T, N, N)

    xw1 = _xw1(struct_features, W1)
    recons0, z = _gcn(adj, xw1, noise, W2)
    s = z.reshape(T, NH)

    s_shift = jnp.concatenate([jnp.zeros((1, NH), jnp.float32), s[:-1]], 0)
    x2 = jnp.concatenate([s_shift, s], 1)
    y0 = _tcn_stage(s, x2, tcn_w0.reshape(2 * NH, NH), tcn_b0.reshape(1, NH))
    y0_shift = jnp.concatenate([jnp.zeros((2, NH), jnp.float32), y0[:-2]], 0)
    x2b = jnp.concatenate([y0_shift, y0], 1)
    y1 = _tcn_stage(y0, x2b, tcn_w1.reshape(2 * NH, NH), tcn_b1.reshape(1, NH))

    out = _outer(recons0, y1.reshape(T, N, H2))
    return out.reshape(2, T, N * N)


# speedup vs baseline: 1.0600x; 1.0600x over previous
import functools

import jax
import jax.numpy as jnp
from jax import lax
from jax.experimental import pallas as pl
from jax.experimental.pallas import tpu as pltpu
from jax.experimental.pallas import tpu_sc as plsc

T = 8
N = 512
F = 128
H1 = 64
H2 = 8
E = 8192
NH = N * H2

_NC = 2
_NS = 16
_TS_PER_CORE = T // _NC
_CHUNKS = _NS // _TS_PER_CORE
_EC = E // _CHUNKS
_A_WORDS = N * N
_SH_WORDS = _TS_PER_CORE * _A_WORDS
_ZW = _SH_WORDS // _NS
_SCAT = 128
_NSCAT = _EC // _SCAT
_NIDX = _EC // 16


_ZB = 8192


def _adj_body(edges, weights, out, src_v, dst_v, w_v, idx2, w2, zbuf, a_sh):
    c = lax.axis_index("c")
    s = lax.axis_index("s")
    ts_local = s // _CHUNKS
    ts = c * _TS_PER_CORE + ts_local
    chunk = s % _CHUNKS

    def zfill(i, _):
        zbuf[pl.ds(i * 16, 16)] = jnp.zeros((16,), jnp.float32)
        return 0

    lax.fori_loop(0, _ZB // 16, zfill, 0)

    def zcopy(i, _):
        pltpu.sync_copy(zbuf, a_sh.at[pl.ds(s * _ZW + i * _ZB, _ZB)])
        return 0

    lax.fori_loop(0, _ZW // _ZB, zcopy, 0)
    pltpu.sync_copy(edges.at[ts, 0, pl.ds(chunk * _EC, _EC)], src_v)
    pltpu.sync_copy(edges.at[ts, 1, pl.ds(chunk * _EC, _EC)], dst_v)
    pltpu.sync_copy(weights.at[ts, pl.ds(chunk * _EC, _EC)], w_v)

    base = ts_local * _A_WORDS

    def idx_body(i, _):
        d = dst_v[pl.ds(i * 16, 16)]
        sr = src_v[pl.ds(i * 16, 16)]
        ww = w_v[pl.ds(i * 16, 16)]
        ix = base + d * N + sr
        row = i // (_SCAT // 16)
        col = (i % (_SCAT // 16)) * 16
        idx2[row, pl.ds(col, 16)] = ix
        w2[row, pl.ds(col, 16)] = ww
        return 0

    lax.fori_loop(0, _NIDX, idx_body, 0)
    plsc.subcore_barrier()

    def scat_body(j, _):
        pltpu.sync_copy(w2.at[j], a_sh.at[idx2.at[j]], add=True)
        return 0

    lax.fori_loop(0, _NSCAT, scat_body, 0)
    plsc.subcore_barrier()

    out_row = c * _TS_PER_CORE + s // _CHUNKS
    out_col = (s % _CHUNKS) * _ZW
    pltpu.sync_copy(a_sh.at[pl.ds(s * _ZW, _ZW)],
                    out.at[out_row, pl.ds(out_col, _ZW)])


def _build_adj(edge_index, edge_weight):
    mesh = plsc.VectorSubcoreMesh(core_axis_name="c", subcore_axis_name="s")
    k = pl.kernel(
        _adj_body,
        mesh=mesh,
        out_type=jax.ShapeDtypeStruct((T, _A_WORDS), jnp.float32),
        scratch_types=[
            pltpu.VMEM((_EC,), jnp.int32),
            pltpu.VMEM((_EC,), jnp.int32),
            pltpu.VMEM((_EC,), jnp.float32),
            pltpu.VMEM((_NSCAT, _SCAT), jnp.int32),
            pltpu.VMEM((_NSCAT, _SCAT), jnp.float32),
            pltpu.VMEM((_ZB,), jnp.float32),
            pltpu.VMEM_SHARED((_SH_WORDS,), jnp.float32),
        ],
    )
    return k(edge_index, edge_weight)


def _xw1_body(x_ref, w1_ref, o_ref):
    o_ref[0] = jnp.dot(x_ref[0], w1_ref[0], preferred_element_type=jnp.float32)


def _xw1(x, W1):
    return pl.pallas_call(
        _xw1_body,
        grid=(T,),
        in_specs=[
            pl.BlockSpec((1, N, F), lambda i: (i, 0, 0)),
            pl.BlockSpec((1, F, H1), lambda i: (i, 0, 0)),
        ],
        out_specs=pl.BlockSpec((1, N, H1), lambda i: (i, 0, 0)),
        out_shape=jax.ShapeDtypeStruct((T, N, H1), jnp.float32),
    )(x, W1)


def _gcn_body(a_ref, xw1_ref, n_ref, w2_ref, rec_ref, z_ref):
    a = a_ref[0]
    h = jnp.maximum(jnp.dot(a, xw1_ref[0], preferred_element_type=jnp.float32),
                    0.0)
    h = h + 0.1 * n_ref[0]
    hw2 = jnp.dot(h, w2_ref[0], preferred_element_type=jnp.float32)
    z = jnp.dot(a, hw2, preferred_element_type=jnp.float32)
    z_ref[0] = z
    rec_ref[0] = lax.dot_general(z, z, (((1,), (1,)), ((), ())),
                                 preferred_element_type=jnp.float32)


def _gcn(adj, xw1, noise, W2):
    return pl.pallas_call(
        _gcn_body,
        grid=(T,),
        in_specs=[
            pl.BlockSpec((1, N, N), lambda i: (i, 0, 0)),
            pl.BlockSpec((1, N, H1), lambda i: (i, 0, 0)),
            pl.BlockSpec((1, N, H1), lambda i: (i, 0, 0)),
            pl.BlockSpec((1, H1, H2), lambda i: (i, 0, 0)),
        ],
        out_specs=[
            pl.BlockSpec((1, N, N), lambda i: (i, 0, 0)),
            pl.BlockSpec((1, N, H2), lambda i: (i, 0, 0)),
        ],
        out_shape=[
            jax.ShapeDtypeStruct((T, N, N), jnp.float32),
            jax.ShapeDtypeStruct((T, N, H2), jnp.float32),
        ],
    )(adj, xw1, noise, W2)


_CB = 256
_NB = NH // _CB


def _conv_block(x, res, w_ref, b, shift):
    tap0 = jnp.dot(x[:T - shift], w_ref[pl.ds(0, NH), :],
                   preferred_element_type=jnp.float32)
    tap0 = jnp.concatenate([jnp.zeros((shift, _CB), jnp.float32), tap0], 0)
    tap1 = jnp.dot(x, w_ref[pl.ds(NH, NH), :],
                   preferred_element_type=jnp.float32)
    y = jnp.maximum(b + tap0 + tap1, 0.0)
    return jnp.maximum(y + res, 0.0)


def _tcn_body(s_ref, w0_ref, w1_ref, b0_ref, b1_ref, y1_ref, y0_scr):
    st = pl.program_id(0)
    i = pl.program_id(1)

    @pl.when(st == 0)
    def _stage0():
        res = s_ref[:, pl.ds(i * _CB, _CB)]
        y0_scr[:, pl.ds(i * _CB, _CB)] = _conv_block(
            s_ref[...], res, w0_ref, b0_ref[...], 1)

    @pl.when(st == 1)
    def _stage1():
        res = y0_scr[:, pl.ds(i * _CB, _CB)]
        y1_ref[...] = _conv_block(y0_scr[...], res, w1_ref, b1_ref[...], 2)


def _tcn(s, w0_cat, w1_cat, b0, b1):
    last = _NB - 1
    return pl.pallas_call(
        _tcn_body,
        grid=(2, _NB),
        in_specs=[
            pl.BlockSpec((T, NH), lambda st, i: (0, 0)),
            pl.BlockSpec((2 * NH, _CB),
                         lambda st, i: (0, jnp.where(st == 0, i, last))),
            pl.BlockSpec((2 * NH, _CB),
                         lambda st, i: (0, jnp.where(st == 0, 0, i))),
            pl.BlockSpec((1, _CB),
                         lambda st, i: (0, jnp.where(st == 0, i, last))),
            pl.BlockSpec((1, _CB),
                         lambda st, i: (0, jnp.where(st == 0, 0, i))),
        ],
        out_specs=pl.BlockSpec((T, _CB), lambda st, i: (0, i)),
        out_shape=jax.ShapeDtypeStruct((T, NH), jnp.float32),
        scratch_shapes=[pltpu.VMEM((T, NH), jnp.float32)],
    )(s, w0_cat, w1_cat, b0, b1)


def _outer_body(rec0_ref, z_ref, r_ref):
    z = z_ref[0]
    r_ref[0, 0] = rec0_ref[0]
    r_ref[1, 0] = lax.dot_general(z, z, (((1,), (1,)), ((), ())),
                                  preferred_element_type=jnp.float32)


def _outer(rec0, z):
    return pl.pallas_call(
        _outer_body,
        grid=(T,),
        in_specs=[
            pl.BlockSpec((1, N, N), lambda i: (i, 0, 0)),
            pl.BlockSpec((1, N, H2), lambda i: (i, 0, 0)),
        ],
        out_specs=pl.BlockSpec((2, 1, N, N), lambda i: (0, i, 0, 0)),
        out_shape=jax.ShapeDtypeStruct((2, T, N, N), jnp.float32),
    )(rec0, z)


def kernel(struct_features, edge_index, edge_weight, noise, W1, W2,
           tcn_w0, tcn_b0, tcn_w1, tcn_b1):
    edge_index = edge_index.astype(jnp.int32)
    edge_weight = edge_weight.astype(jnp.float32)

    adj = _build_adj(edge_index, edge_weight).reshape(T, N, N)

    xw1 = _xw1(struct_features, W1)
    recons0, z = _gcn(adj, xw1, noise, W2)
    s = z.reshape(T, NH)

    y1 = _tcn(s, tcn_w0.reshape(2 * NH, NH), tcn_w1.reshape(2 * NH, NH),
              tcn_b0.reshape(1, NH), tcn_b1.reshape(1, NH))

    out = _outer(recons0, y1.reshape(T, N, H2))
    return out.reshape(2, T, N * N)


# speedup vs baseline: 1.0875x; 1.0259x over previous
import functools

import jax
import jax.numpy as jnp
from jax import lax
from jax.experimental import pallas as pl
from jax.experimental.pallas import tpu as pltpu
from jax.experimental.pallas import tpu_sc as plsc

T = 8
N = 512
F = 128
H1 = 64
H2 = 8
E = 8192
NH = N * H2

_NC = 2
_NS = 16
_TS_PER_CORE = T // _NC
_CHUNKS = _NS // _TS_PER_CORE
_EC = E // _CHUNKS
_A_WORDS = N * N
_SH_WORDS = _TS_PER_CORE * _A_WORDS
_ZW = _SH_WORDS // _NS
_SCAT = 128
_NSCAT = _EC // _SCAT
_NIDX = _EC // 16


_ZB = 8192


def _adj_body(edges, weights, out, src_v, dst_v, w_v, idx2, w2, zbuf, a_sh):
    c = lax.axis_index("c")
    s = lax.axis_index("s")
    ts_local = s // _CHUNKS
    ts = c * _TS_PER_CORE + ts_local
    chunk = s % _CHUNKS

    def zfill(i, _):
        zbuf[pl.ds(i * 16, 16)] = jnp.zeros((16,), jnp.float32)
        return 0

    lax.fori_loop(0, _ZB // 16, zfill, 0)

    def zcopy(i, _):
        pltpu.sync_copy(zbuf, a_sh.at[pl.ds(s * _ZW + i * _ZB, _ZB)])
        return 0

    lax.fori_loop(0, _ZW // _ZB, zcopy, 0)
    pltpu.sync_copy(edges.at[ts, 0, pl.ds(chunk * _EC, _EC)], src_v)
    pltpu.sync_copy(edges.at[ts, 1, pl.ds(chunk * _EC, _EC)], dst_v)
    pltpu.sync_copy(weights.at[ts, pl.ds(chunk * _EC, _EC)], w_v)

    base = ts_local * _A_WORDS

    def idx_body(i, _):
        d = dst_v[pl.ds(i * 16, 16)]
        sr = src_v[pl.ds(i * 16, 16)]
        ww = w_v[pl.ds(i * 16, 16)]
        ix = base + d * N + sr
        row = i // (_SCAT // 16)
        col = (i % (_SCAT // 16)) * 16
        idx2[row, pl.ds(col, 16)] = ix
        w2[row, pl.ds(col, 16)] = ww
        return 0

    lax.fori_loop(0, _NIDX, idx_body, 0)
    plsc.subcore_barrier()

    def scat_body(j, _):
        pltpu.sync_copy(w2.at[j], a_sh.at[idx2.at[j]], add=True)
        return 0

    lax.fori_loop(0, _NSCAT, scat_body, 0)
    plsc.subcore_barrier()

    out_row = c * _TS_PER_CORE + s // _CHUNKS
    out_col = (s % _CHUNKS) * _ZW
    pltpu.sync_copy(a_sh.at[pl.ds(s * _ZW, _ZW)],
                    out.at[out_row, pl.ds(out_col, _ZW)])


def _build_adj(edge_index, edge_weight):
    mesh = plsc.VectorSubcoreMesh(core_axis_name="c", subcore_axis_name="s")
    k = pl.kernel(
        _adj_body,
        mesh=mesh,
        out_type=jax.ShapeDtypeStruct((T, _A_WORDS), jnp.float32),
        scratch_types=[
            pltpu.VMEM((_EC,), jnp.int32),
            pltpu.VMEM((_EC,), jnp.int32),
            pltpu.VMEM((_EC,), jnp.float32),
            pltpu.VMEM((_NSCAT, _SCAT), jnp.int32),
            pltpu.VMEM((_NSCAT, _SCAT), jnp.float32),
            pltpu.VMEM((_ZB,), jnp.float32),
            pltpu.VMEM_SHARED((_SH_WORDS,), jnp.float32),
        ],
    )
    return k(edge_index, edge_weight)


def _xw1_body(x_ref, w1_ref, o_ref):
    o_ref[0] = jnp.dot(x_ref[0], w1_ref[0], preferred_element_type=jnp.float32)


def _xw1(x, W1):
    return pl.pallas_call(
        _xw1_body,
        grid=(T,),
        in_specs=[
            pl.BlockSpec((1, N, F), lambda i: (i, 0, 0)),
            pl.BlockSpec((1, F, H1), lambda i: (i, 0, 0)),
        ],
        out_specs=pl.BlockSpec((1, N, H1), lambda i: (i, 0, 0)),
        out_shape=jax.ShapeDtypeStruct((T, N, H1), jnp.float32),
    )(x, W1)


def _gcn_body(a_ref, xw1_ref, n_ref, w2_ref, z_ref):
    a = a_ref[0]
    h = jnp.maximum(jnp.dot(a, xw1_ref[0], preferred_element_type=jnp.float32),
                    0.0)
    h = h + 0.1 * n_ref[0]
    hw2 = jnp.dot(h, w2_ref[0], preferred_element_type=jnp.float32)
    z_ref[0] = jnp.dot(a, hw2, preferred_element_type=jnp.float32)


def _gcn(adj, xw1, noise, W2):
    return pl.pallas_call(
        _gcn_body,
        grid=(T,),
        in_specs=[
            pl.BlockSpec((1, N, N), lambda i: (i, 0, 0)),
            pl.BlockSpec((1, N, H1), lambda i: (i, 0, 0)),
            pl.BlockSpec((1, N, H1), lambda i: (i, 0, 0)),
            pl.BlockSpec((1, H1, H2), lambda i: (i, 0, 0)),
        ],
        out_specs=pl.BlockSpec((1, N, H2), lambda i: (i, 0, 0)),
        out_shape=jax.ShapeDtypeStruct((T, N, H2), jnp.float32),
    )(adj, xw1, noise, W2)


_CB = 256
_NB = NH // _CB


def _conv_block(x, res, w_ref, b, shift):
    tap0 = jnp.dot(x[:T - shift], w_ref[pl.ds(0, NH), :],
                   preferred_element_type=jnp.float32)
    tap0 = jnp.concatenate([jnp.zeros((shift, _CB), jnp.float32), tap0], 0)
    tap1 = jnp.dot(x, w_ref[pl.ds(NH, NH), :],
                   preferred_element_type=jnp.float32)
    y = jnp.maximum(b + tap0 + tap1, 0.0)
    return jnp.maximum(y + res, 0.0)


def _tcn_body(s_ref, w0_ref, w1_ref, b0_ref, b1_ref, y1_ref, y0_scr):
    st = pl.program_id(0)
    i = pl.program_id(1)

    @pl.when(st == 0)
    def _stage0():
        res = s_ref[:, pl.ds(i * _CB, _CB)]
        y0_scr[:, pl.ds(i * _CB, _CB)] = _conv_block(
            s_ref[...], res, w0_ref, b0_ref[...], 1)

    @pl.when(st == 1)
    def _stage1():
        res = y0_scr[:, pl.ds(i * _CB, _CB)]
        y1_ref[...] = _conv_block(y0_scr[...], res, w1_ref, b1_ref[...], 2)


def _tcn(s, w0_cat, w1_cat, b0, b1):
    last = _NB - 1
    return pl.pallas_call(
        _tcn_body,
        grid=(2, _NB),
        in_specs=[
            pl.BlockSpec((T, NH), lambda st, i: (0, 0)),
            pl.BlockSpec((2 * NH, _CB),
                         lambda st, i: (0, jnp.where(st == 0, i, last))),
            pl.BlockSpec((2 * NH, _CB),
                         lambda st, i: (0, jnp.where(st == 0, 0, i))),
            pl.BlockSpec((1, _CB),
                         lambda st, i: (0, jnp.where(st == 0, i, last))),
            pl.BlockSpec((1, _CB),
                         lambda st, i: (0, jnp.where(st == 0, 0, i))),
        ],
        out_specs=pl.BlockSpec((T, _CB), lambda st, i: (0, i)),
        out_shape=jax.ShapeDtypeStruct((T, NH), jnp.float32),
        scratch_shapes=[pltpu.VMEM((T, NH), jnp.float32)],
    )(s, w0_cat, w1_cat, b0, b1)


def _outer_body(z_ref, y_ref, r_ref):
    z = z_ref[0]
    y = y_ref[0]
    r_ref[0, 0] = lax.dot_general(z, z, (((1,), (1,)), ((), ())),
                                  preferred_element_type=jnp.float32)
    r_ref[1, 0] = lax.dot_general(y, y, (((1,), (1,)), ((), ())),
                                  preferred_element_type=jnp.float32)


def _outer(z, y1):
    return pl.pallas_call(
        _outer_body,
        grid=(T,),
        in_specs=[
            pl.BlockSpec((1, N, H2), lambda i: (i, 0, 0)),
            pl.BlockSpec((1, N, H2), lambda i: (i, 0, 0)),
        ],
        out_specs=pl.BlockSpec((2, 1, N, N), lambda i: (0, i, 0, 0)),
        out_shape=jax.ShapeDtypeStruct((2, T, N, N), jnp.float32),
    )(z, y1)


def kernel(struct_features, edge_index, edge_weight, noise, W1, W2,
           tcn_w0, tcn_b0, tcn_w1, tcn_b1):
    edge_index = edge_index.astype(jnp.int32)
    edge_weight = edge_weight.astype(jnp.float32)

    adj = _build_adj(edge_index, edge_weight).reshape(T, N, N)

    xw1 = _xw1(struct_features, W1)
    z = _gcn(adj, xw1, noise, W2)
    s = z.reshape(T, NH)

    y1 = _tcn(s, tcn_w0.reshape(2 * NH, NH), tcn_w1.reshape(2 * NH, NH),
              tcn_b0.reshape(1, NH), tcn_b1.reshape(1, NH))

    out = _outer(z, y1.reshape(T, N, H2))
    return out.reshape(2, T, N * N)
